# baseline (speedup 1.0000x reference)
import jax
import jax.numpy as jnp
from jax import lax
from jax.experimental import pallas as pl
from jax.experimental.pallas import tpu as pltpu
from jax.experimental.pallas import tpu_sc as plsc

N_LIG = 2048
N_PROT = 16384
N_BATCH = 32
D2_MAX = 4.5 * 4.5
K_NBR = 15
EPS = 1e-8

NC = 2
NW = NC * 16
QPW = N_LIG // NW
L = 16
QN = 4
SMAX = 1024

_INF = float("inf")


def _sqrt16(x):
    xi = plsc.bitcast(x, jnp.int32)
    y = plsc.bitcast(jnp.int32(0x5F3759DF) - (xi >> 1), jnp.float32)
    for _ in range(3):
        y = y * (1.5 - 0.5 * x * y * y)
    return x * y


def _sc_body(lgx_h, lgy_h, lgz_h, ltx_h, lty_h, ltz_h,
             pgx_h, pgy_h, pgz_h, ptx_h, pty_h, ptz_h,
             lb_hbm, pb_hbm, tw_hbm,
             se_hbm, cnt_hbm,
             lgx, lgy, lgz, ltx, lty, ltz, lb, tww, cnts,
             wtx, wty, wtz, wgx, wgy, wgz,
             pb, avbuf, apbuf, ovec, dsem, psem, asem, bsem):
    wid = lax.axis_index("s") * NC + lax.axis_index("c")
    base = wid * QPW

    b1 = pl.multiple_of(base, QPW)
    pre = [
        pltpu.async_copy(pb_hbm, pb, psem),
        pltpu.async_copy(lb_hbm.at[pl.ds(b1, QPW)], lb, psem),
    ]
    bulk = [
        pltpu.async_copy(lgx_h.at[pl.ds(b1, QPW)], lgx, dsem),
        pltpu.async_copy(lgy_h.at[pl.ds(b1, QPW)], lgy, dsem),
        pltpu.async_copy(lgz_h.at[pl.ds(b1, QPW)], lgz, dsem),
        pltpu.async_copy(ltx_h.at[pl.ds(b1, QPW)], ltx, dsem),
        pltpu.async_copy(lty_h.at[pl.ds(b1, QPW)], lty, dsem),
        pltpu.async_copy(ltz_h.at[pl.ds(b1, QPW)], ltz, dsem),
        pltpu.async_copy(tw_hbm, tww, dsem),
    ]

    lane = lax.iota(jnp.int32, L)
    inf16 = jnp.full((L,), _INF)
    zeroi = jnp.zeros((L,), jnp.int32)
    zero16 = jnp.zeros((L,), jnp.float32)

    def initq(q, _):
        avbuf[pl.ds(q * L, L)] = inf16
        apbuf[pl.ds(q * L, L)] = zeroi
        return 0
    lax.fori_loop(0, QPW, initq, 0)

    for c in pre:
        c.wait()

    for g in range(3):
        bvec = lane + g * L
        lo = jnp.zeros((L,), jnp.int32)
        p = N_PROT // 2
        while p >= 1:
            v = plsc.load_gather(pb, [lo + (p - 1)])
            lo = jnp.where(v < bvec, lo + p, lo)
            p //= 2
        v = plsc.load_gather(pb, [lo])
        lo = jnp.where(v < bvec, lo + 1, lo)
        cnts[pl.ds(g * L, L)] = lo

    bfirst = plsc.load_gather(lb, [zeroi])
    blast = plsc.load_gather(lb, [zeroi + (QPW - 1)])
    slo = plsc.load_gather(cnts, [bfirst])[0] & ~(L - 1)
    shi = plsc.load_gather(cnts, [blast + 1])[0]
    nwin = (shi - slo + (SMAX - 1)) >> 10

    bend0 = jnp.minimum(slo + SMAX, shi)
    ws0 = pl.multiple_of(jnp.minimum(slo, N_PROT - SMAX), L)
    a0 = [
        pltpu.async_copy(ptx_h.at[pl.ds(ws0, SMAX)], wtx, asem),
        pltpu.async_copy(pty_h.at[pl.ds(ws0, SMAX)], wty, asem),
        pltpu.async_copy(ptz_h.at[pl.ds(ws0, SMAX)], wtz, asem),
    ]
    b0 = [
        pltpu.async_copy(pgx_h.at[pl.ds(ws0, SMAX)], wgx, bsem),
        pltpu.async_copy(pgy_h.at[pl.ds(ws0, SMAX)], wgy, bsem),
        pltpu.async_copy(pgz_h.at[pl.ds(ws0, SMAX)], wgz, bsem),
    ]
    for c in bulk:
        c.wait()
    for c in a0:
        c.wait()

    def ascan(bstart, bend, ws):
        def qbody(q, _):
            iq = [jnp.full((L,), QN * q + k, jnp.int32) for k in range(QN)]
            bq = [plsc.load_gather(lb, [i]) for i in iq]
            sq = [plsc.load_gather(cnts, [b]) for b in bq]
            eq = [plsc.load_gather(cnts, [b + 1]) for b in bq]
            ltxq = [plsc.load_gather(ltx, [i]) for i in iq]
            ltyq = [plsc.load_gather(lty, [i]) for i in iq]
            ltzq = [plsc.load_gather(ltz, [i]) for i in iq]
            ps = jnp.maximum(sq[0][0] & ~(L - 1), bstart)
            pe = jnp.minimum(eq[-1][0], bend)
            nblk = jnp.maximum(pe - ps + (L - 1), 0) >> 4

            avs = [avbuf[pl.ds((QN * q + k) * L, L)] for k in range(QN)]
            aps = [apbuf[pl.ds((QN * q + k) * L, L)] for k in range(QN)]

            def tblock(t, c2):
                a, p_ = list(c2[:QN]), list(c2[QN:])
                j0 = ps + t * L
                posv = lane + j0
                lj = j0 - ws
                px = wtx[pl.ds(lj, L)]
                py = wty[pl.ds(lj, L)]
                pz = wtz[pl.ds(lj, L)]
                for k in range(QN):
                    dx = px - ltxq[k]
                    dy = py - ltyq[k]
                    dz = pz - ltzq[k]
                    d2 = dx * dx + dy * dy + dz * dz
                    d2 = jnp.where((posv >= sq[k]) & (posv < eq[k]), d2, _INF)
                    bv, bp = plsc.sort_key_val(d2, posv, descending=True)
                    tk = bv < a[k]
                    mv = jnp.where(tk, bv, a[k])
                    mp = jnp.where(tk, bp, p_[k])
                    a[k], p_[k] = plsc.sort_key_val(mv, mp)
                return tuple(a) + tuple(p_)

            res = lax.fori_loop(0, nblk, tblock, tuple(avs) + tuple(aps))
            for k in range(QN):
                avbuf[pl.ds((QN * q + k) * L, L)] = res[k]
                apbuf[pl.ds((QN * q + k) * L, L)] = res[QN + k]
            return 0

        lax.fori_loop(0, QPW // QN, qbody, 0)
        return 0

    ascan(slo, bend0, ws0)

    def awin(w, _):
        bstart = slo + w * SMAX
        bend = jnp.minimum(bstart + SMAX, shi)
        ws = pl.multiple_of(jnp.minimum(bstart, N_PROT - SMAX), L)
        cs = [
            pltpu.async_copy(ptx_h.at[pl.ds(ws, SMAX)], wtx, asem),
            pltpu.async_copy(pty_h.at[pl.ds(ws, SMAX)], wty, asem),
            pltpu.async_copy(ptz_h.at[pl.ds(ws, SMAX)], wtz, asem),
        ]
        for c in cs:
            c.wait()
        return ascan(bstart, bend, ws)

    lax.fori_loop(1, nwin, awin, 0)

    def bscan(bstart, bend, ws, carry):
        def qfin(q, carry2):
            ca2, na2 = carry2
            isplat = jnp.full((L,), q, jnp.int32)
            av = avbuf[pl.ds(q * L, L)]
            ap = apbuf[pl.ds(q * L, L)]
            bq = plsc.load_gather(lb, [isplat])
            twv = plsc.load_gather(tww, [bq])
            inwin = (ap >= bstart) & (ap < bend)
            lp = jnp.where(inwin, ap - ws, 0)
            gx = plsc.load_gather(lgx, [isplat]) - plsc.load_gather(wgx, [lp])
            gy = plsc.load_gather(lgy, [isplat]) - plsc.load_gather(wgy, [lp])
            gz = plsc.load_gather(lgz, [isplat]) - plsc.load_gather(wgz, [lp])
            d2g = gx * gx + gy * gy + gz * gz
            d2t = jnp.minimum(av, 1e8)
            dij_g = _sqrt16(jnp.maximum(d2g, EPS))
            dij_t = _sqrt16(jnp.maximum(d2t, EPS))
            se = (dij_g - dij_t) * (dij_g - dij_t)
            validm = (av <= D2_MAX) & (lane < K_NBR) & inwin
            ca2 = ca2 + jnp.where(validm, se * twv, 0.0)
            na2 = na2 + jnp.where(validm, 1.0, 0.0)
            return ca2, na2

        return lax.fori_loop(0, QPW, qfin, carry)

    for c in b0:
        c.wait()
    carry0 = bscan(slo, bend0, ws0, (zero16, zero16))

    def bwin(w, carry):
        bstart = slo + w * SMAX
        bend = jnp.minimum(bstart + SMAX, shi)
        ws = pl.multiple_of(jnp.minimum(bstart, N_PROT - SMAX), L)
        cs = [
            pltpu.async_copy(pgx_h.at[pl.ds(ws, SMAX)], wgx, bsem),
            pltpu.async_copy(pgy_h.at[pl.ds(ws, SMAX)], wgy, bsem),
            pltpu.async_copy(pgz_h.at[pl.ds(ws, SMAX)], wgz, bsem),
        ]
        for c in cs:
            c.wait()
        return bscan(bstart, bend, ws, carry)

    ca, na = lax.fori_loop(1, nwin, bwin, carry0)
    ovec[...] = ca
    pltpu.sync_copy(ovec, se_hbm.at[wid])
    ovec[...] = na
    pltpu.sync_copy(ovec, cnt_hbm.at[wid])


@jax.jit
def _run(lgx, lgy, lgz, ltx, lty, ltz, pgx, pgy, pgz, ptx, pty, ptz,
         lb, pb, tw):
    mesh = plsc.VectorSubcoreMesh(core_axis_name="c", subcore_axis_name="s",
                                  num_cores=NC, num_subcores=16)
    f32, i32 = jnp.float32, jnp.int32

    call = pl.kernel(
        _sc_body,
        out_type=(
            jax.ShapeDtypeStruct((NW, L), f32),
            jax.ShapeDtypeStruct((NW, L), f32),
        ),
        mesh=mesh,
        compiler_params=pltpu.CompilerParams(needs_layout_passes=False),
        scratch_types=(
            pltpu.VMEM((QPW,), f32), pltpu.VMEM((QPW,), f32),
            pltpu.VMEM((QPW,), f32), pltpu.VMEM((QPW,), f32),
            pltpu.VMEM((QPW,), f32), pltpu.VMEM((QPW,), f32),
            pltpu.VMEM((QPW,), i32),
            pltpu.VMEM((N_BATCH,), f32),
            pltpu.VMEM((3 * L,), i32),
            pltpu.VMEM((SMAX,), f32), pltpu.VMEM((SMAX,), f32),
            pltpu.VMEM((SMAX,), f32),
            pltpu.VMEM((SMAX,), f32), pltpu.VMEM((SMAX,), f32),
            pltpu.VMEM((SMAX,), f32),
            pltpu.VMEM((N_PROT,), i32),
            pltpu.VMEM((QPW * L,), f32),
            pltpu.VMEM((QPW * L,), i32),
            pltpu.VMEM((L,), f32),
            pltpu.SemaphoreType.DMA, pltpu.SemaphoreType.DMA,
            pltpu.SemaphoreType.DMA, pltpu.SemaphoreType.DMA,
        ),
    )
    se0, cnt0 = call(lgx, lgy, lgz, ltx, lty, ltz, pgx, pgy, pgz,
                     ptx, pty, ptz, lb, pb, tw)
    return jnp.sum(se0) / jnp.maximum(jnp.sum(cnt0), 1.0)


def kernel(lig_x_gen, prot_x_gen, lig_x_true, prot_x_true, lig_batch,
           prot_batch, time_weights):
    pg = [prot_x_gen[:, c] for c in range(3)]
    pt = [prot_x_true[:, c] for c in range(3)]
    lg = [lig_x_gen[:, c] for c in range(3)]
    lt = [lig_x_true[:, c] for c in range(3)]
    return _run(
        *lg, *lt, *pg, *pt,
        lig_batch.astype(jnp.int32), prot_batch.astype(jnp.int32),
        time_weights,
    )


# speedup vs baseline: 1.0710x; 1.0710x over previous
import jax
import jax.numpy as jnp
from jax import lax
from jax.experimental import pallas as pl
from jax.experimental.pallas import tpu as pltpu
from jax.experimental.pallas import tpu_sc as plsc

N_LIG = 2048
N_PROT = 16384
N_BATCH = 32
D2_MAX = 4.5 * 4.5
K_NBR = 15
EPS = 1e-8

NC = 2
NW = NC * 16
QPW = N_LIG // NW
L = 16
QN = 4
SMAX = 2048

_INF = float("inf")


def _sqrt16(x):
    xi = plsc.bitcast(x, jnp.int32)
    y = plsc.bitcast(jnp.int32(0x5F3759DF) - (xi >> 1), jnp.float32)
    for _ in range(3):
        y = y * (1.5 - 0.5 * x * y * y)
    return x * y


def _sc_body(lgx_h, lgy_h, lgz_h, ltx_h, lty_h, ltz_h,
             pgx_h, pgy_h, pgz_h, ptx_h, pty_h, ptz_h,
             lb_hbm, pb_hbm, tw_hbm,
             se_hbm, cnt_hbm,
             lgx, lgy, lgz, ltx, lty, ltz, lb, tww, cnts,
             wtx, wty, wtz, wgx, wgy, wgz,
             pb, avbuf, apbuf, ovec, dsem, psem, asem, bsem):
    wid = lax.axis_index("s") * NC + lax.axis_index("c")
    base = wid * QPW

    b1 = pl.multiple_of(base, QPW)
    pre = [
        pltpu.async_copy(pb_hbm, pb, psem),
        pltpu.async_copy(lb_hbm.at[pl.ds(b1, QPW)], lb, psem),
    ]
    bulk = [
        pltpu.async_copy(lgx_h.at[pl.ds(b1, QPW)], lgx, dsem),
        pltpu.async_copy(lgy_h.at[pl.ds(b1, QPW)], lgy, dsem),
        pltpu.async_copy(lgz_h.at[pl.ds(b1, QPW)], lgz, dsem),
        pltpu.async_copy(ltx_h.at[pl.ds(b1, QPW)], ltx, dsem),
        pltpu.async_copy(lty_h.at[pl.ds(b1, QPW)], lty, dsem),
        pltpu.async_copy(ltz_h.at[pl.ds(b1, QPW)], ltz, dsem),
        pltpu.async_copy(tw_hbm, tww, dsem),
    ]

    lane = lax.iota(jnp.int32, L)
    inf16 = jnp.full((L,), _INF)
    zeroi = jnp.zeros((L,), jnp.int32)
    zero16 = jnp.zeros((L,), jnp.float32)

    def initq(q, _):
        avbuf[pl.ds(q * L, L)] = inf16
        apbuf[pl.ds(q * L, L)] = zeroi
        return 0
    lax.fori_loop(0, QPW, initq, 0)

    for c in pre:
        c.wait()

    for g in range(2):
        bvec = lane + g * L
        lo = jnp.zeros((L,), jnp.int32)
        p = N_PROT // 2
        while p >= 1:
            v = plsc.load_gather(pb, [lo + (p - 1)])
            lo = jnp.where(v < bvec, lo + p, lo)
            p //= 2
        v = plsc.load_gather(pb, [lo])
        lo = jnp.where(v < bvec, lo + 1, lo)
        cnts[pl.ds(g * L, L)] = lo
    cnts[pl.ds(2 * L, L)] = jnp.full((L,), N_PROT, jnp.int32)

    bfirst = plsc.load_gather(lb, [zeroi])
    blast = plsc.load_gather(lb, [zeroi + (QPW - 1)])
    slo = plsc.load_gather(cnts, [bfirst])[0] & ~(L - 1)
    shi = plsc.load_gather(cnts, [blast + 1])[0]
    nwin = (shi - slo + (SMAX - 1)) >> 11

    bend0 = jnp.minimum(slo + SMAX, shi)
    ws0 = pl.multiple_of(jnp.minimum(slo, N_PROT - SMAX), L)
    a0 = [
        pltpu.async_copy(ptx_h.at[pl.ds(ws0, SMAX)], wtx, asem),
        pltpu.async_copy(pty_h.at[pl.ds(ws0, SMAX)], wty, asem),
        pltpu.async_copy(ptz_h.at[pl.ds(ws0, SMAX)], wtz, asem),
    ]
    b0 = [
        pltpu.async_copy(pgx_h.at[pl.ds(ws0, SMAX)], wgx, bsem),
        pltpu.async_copy(pgy_h.at[pl.ds(ws0, SMAX)], wgy, bsem),
        pltpu.async_copy(pgz_h.at[pl.ds(ws0, SMAX)], wgz, bsem),
    ]
    for c in bulk:
        c.wait()
    for c in a0:
        c.wait()

    def ascan(bstart, bend, ws):
        def qbody(q, _):
            iq = [jnp.full((L,), QN * q + k, jnp.int32) for k in range(QN)]
            bq = [plsc.load_gather(lb, [i]) for i in iq]
            sq = [plsc.load_gather(cnts, [b]) for b in bq]
            eq = [plsc.load_gather(cnts, [b + 1]) for b in bq]
            ltxq = [plsc.load_gather(ltx, [i]) for i in iq]
            ltyq = [plsc.load_gather(lty, [i]) for i in iq]
            ltzq = [plsc.load_gather(ltz, [i]) for i in iq]
            ps = jnp.maximum(sq[0][0] & ~(L - 1), bstart)
            pe = jnp.minimum(eq[-1][0], bend)
            nblk = jnp.maximum(pe - ps + (L - 1), 0) >> 4

            avs = [avbuf[pl.ds((QN * q + k) * L, L)] for k in range(QN)]
            aps = [apbuf[pl.ds((QN * q + k) * L, L)] for k in range(QN)]

            def tblock(t, c2):
                a, p_ = list(c2[:QN]), list(c2[QN:])
                j0 = ps + t * L
                posv = lane + j0
                lj = j0 - ws
                px = wtx[pl.ds(lj, L)]
                py = wty[pl.ds(lj, L)]
                pz = wtz[pl.ds(lj, L)]
                for k in range(QN):
                    dx = px - ltxq[k]
                    dy = py - ltyq[k]
                    dz = pz - ltzq[k]
                    d2 = dx * dx + dy * dy + dz * dz
                    d2 = jnp.where((posv >= sq[k]) & (posv < eq[k]), d2, _INF)
                    bv, bp = plsc.sort_key_val(d2, posv, descending=True)
                    tk = bv < a[k]
                    mv = jnp.where(tk, bv, a[k])
                    mp = jnp.where(tk, bp, p_[k])
                    a[k], p_[k] = plsc.sort_key_val(mv, mp)
                return tuple(a) + tuple(p_)

            res = lax.fori_loop(0, nblk, tblock, tuple(avs) + tuple(aps))
            for k in range(QN):
                avbuf[pl.ds((QN * q + k) * L, L)] = res[k]
                apbuf[pl.ds((QN * q + k) * L, L)] = res[QN + k]
            return 0

        lax.fori_loop(0, QPW // QN, qbody, 0)
        return 0

    ascan(slo, bend0, ws0)

    def awin(w, _):
        bstart = slo + w * SMAX
        bend = jnp.minimum(bstart + SMAX, shi)
        ws = pl.multiple_of(jnp.minimum(bstart, N_PROT - SMAX), L)
        cs = [
            pltpu.async_copy(ptx_h.at[pl.ds(ws, SMAX)], wtx, asem),
            pltpu.async_copy(pty_h.at[pl.ds(ws, SMAX)], wty, asem),
            pltpu.async_copy(ptz_h.at[pl.ds(ws, SMAX)], wtz, asem),
        ]
        for c in cs:
            c.wait()
        return ascan(bstart, bend, ws)

    lax.fori_loop(1, nwin, awin, 0)

    def bscan(bstart, bend, ws, carry):
        def qfin(q, carry2):
            ca2, na2 = carry2
            isplat = jnp.full((L,), q, jnp.int32)
            av = avbuf[pl.ds(q * L, L)]
            ap = apbuf[pl.ds(q * L, L)]
            bq = plsc.load_gather(lb, [isplat])
            twv = plsc.load_gather(tww, [bq])
            inwin = (ap >= bstart) & (ap < bend)
            lp = jnp.where(inwin, ap - ws, 0)
            gx = plsc.load_gather(lgx, [isplat]) - plsc.load_gather(wgx, [lp])
            gy = plsc.load_gather(lgy, [isplat]) - plsc.load_gather(wgy, [lp])
            gz = plsc.load_gather(lgz, [isplat]) - plsc.load_gather(wgz, [lp])
            d2g = gx * gx + gy * gy + gz * gz
            d2t = jnp.minimum(av, 1e8)
            dij_g = _sqrt16(jnp.maximum(d2g, EPS))
            dij_t = _sqrt16(jnp.maximum(d2t, EPS))
            se = (dij_g - dij_t) * (dij_g - dij_t)
            validm = (av <= D2_MAX) & (lane < K_NBR) & inwin
            ca2 = ca2 + jnp.where(validm, se * twv, 0.0)
            na2 = na2 + jnp.where(validm, 1.0, 0.0)
            return ca2, na2

        return lax.fori_loop(0, QPW, qfin, carry)

    for c in b0:
        c.wait()
    carry0 = bscan(slo, bend0, ws0, (zero16, zero16))

    def bwin(w, carry):
        bstart = slo + w * SMAX
        bend = jnp.minimum(bstart + SMAX, shi)
        ws = pl.multiple_of(jnp.minimum(bstart, N_PROT - SMAX), L)
        cs = [
            pltpu.async_copy(pgx_h.at[pl.ds(ws, SMAX)], wgx, bsem),
            pltpu.async_copy(pgy_h.at[pl.ds(ws, SMAX)], wgy, bsem),
            pltpu.async_copy(pgz_h.at[pl.ds(ws, SMAX)], wgz, bsem),
        ]
        for c in cs:
            c.wait()
        return bscan(bstart, bend, ws, carry)

    ca, na = lax.fori_loop(1, nwin, bwin, carry0)
    ovec[...] = ca
    pltpu.sync_copy(ovec, se_hbm.at[wid])
    ovec[...] = na
    pltpu.sync_copy(ovec, cnt_hbm.at[wid])


@jax.jit
def _run(lgx, lgy, lgz, ltx, lty, ltz, pgx, pgy, pgz, ptx, pty, ptz,
         lb, pb, tw):
    mesh = plsc.VectorSubcoreMesh(core_axis_name="c", subcore_axis_name="s",
                                  num_cores=NC, num_subcores=16)
    f32, i32 = jnp.float32, jnp.int32

    call = pl.kernel(
        _sc_body,
        out_type=(
            jax.ShapeDtypeStruct((NW, L), f32),
            jax.ShapeDtypeStruct((NW, L), f32),
        ),
        mesh=mesh,
        compiler_params=pltpu.CompilerParams(needs_layout_passes=False),
        scratch_types=(
            pltpu.VMEM((QPW,), f32), pltpu.VMEM((QPW,), f32),
            pltpu.VMEM((QPW,), f32), pltpu.VMEM((QPW,), f32),
            pltpu.VMEM((QPW,), f32), pltpu.VMEM((QPW,), f32),
            pltpu.VMEM((QPW,), i32),
            pltpu.VMEM((N_BATCH,), f32),
            pltpu.VMEM((3 * L,), i32),
            pltpu.VMEM((SMAX,), f32), pltpu.VMEM((SMAX,), f32),
            pltpu.VMEM((SMAX,), f32),
            pltpu.VMEM((SMAX,), f32), pltpu.VMEM((SMAX,), f32),
            pltpu.VMEM((SMAX,), f32),
            pltpu.VMEM((N_PROT,), i32),
            pltpu.VMEM((QPW * L,), f32),
            pltpu.VMEM((QPW * L,), i32),
            pltpu.VMEM((L,), f32),
            pltpu.SemaphoreType.DMA, pltpu.SemaphoreType.DMA,
            pltpu.SemaphoreType.DMA, pltpu.SemaphoreType.DMA,
        ),
    )
    se0, cnt0 = call(lgx, lgy, lgz, ltx, lty, ltz, pgx, pgy, pgz,
                     ptx, pty, ptz, lb, pb, tw)
    return jnp.sum(se0) / jnp.maximum(jnp.sum(cnt0), 1.0)


def kernel(lig_x_gen, prot_x_gen, lig_x_true, prot_x_true, lig_batch,
           prot_batch, time_weights):
    pg = [prot_x_gen[:, c] for c in range(3)]
    pt = [prot_x_true[:, c] for c in range(3)]
    lg = [lig_x_gen[:, c] for c in range(3)]
    lt = [lig_x_true[:, c] for c in range(3)]
    return _run(
        *lg, *lt, *pg, *pt,
        lig_batch.astype(jnp.int32), prot_batch.astype(jnp.int32),
        time_weights,
    )
